# Initial kernel scaffold; baseline (speedup 1.0000x reference)
import jax
import jax.numpy as jnp
from jax.experimental import pallas as pl


def kernel(x, edge_index, W1, attn_l1, attn_r1, b1, W2, attn_l2, attn_r2, b2):
    raise NotImplementedError("write your pallas kernel here")


# speedup vs baseline: 52.1417x; 52.1417x over previous
import jax
import jax.numpy as jnp
from jax import lax
from jax.experimental import pallas as pl
from jax.experimental.pallas import tpu as pltpu
from jax.experimental.pallas import tpu_sc as plsc

N = 10000
E = 640000
H = 4
D_HID = 128
N_CLASSES = 64
DT = 128

C = 100
ROWS = E // C
TROWS = ROWS // 32
GRP = 8
NGRP = TROWS // GRP
OFFS = (0, 16, 32, 48, 64, 80, 84)
NEG_SLOPE = 0.2


def _attn_outputs(z_cols, al_r, ar_r, dh):
    els, ers = [], []
    for h in range(H):
        zh = z_cols[:, h * dh:(h + 1) * dh]
        els.append(jnp.sum(zh * al_r[h, :][None, :], axis=1))
        ers.append(jnp.sum(zh * ar_r[h, :][None, :], axis=1))
    ms = [jnp.maximum(t, NEG_SLOPE * t)
          for t in [jnp.max(els[h]) + jnp.max(ers[h]) for h in range(H)]]
    mbc = jnp.broadcast_to(jnp.stack(ms, axis=0)[:, None], (H, 16))
    return jnp.stack(els, axis=0), jnp.stack(ers, axis=0), mbc


def _tc1_body(x_r, w_r, al_r, ar_r, z0_r, z1_r, z2_r, z3_r,
              el_r, er_r, m_r):
    z = jnp.dot(x_r[...], w_r[...], preferred_element_type=jnp.float32)
    zrefs = (z0_r, z1_r, z2_r, z3_r)
    for h in range(H):
        zrefs[h][...] = z[:, h * D_HID:(h + 1) * D_HID]
    el_r[...], er_r[...], m_r[...] = _attn_outputs(z, al_r, ar_r, D_HID)


def _tc2a_body(p0_r, p1_r, p2_r, p3_r, b1_r, w2_r, zp01_r, zp23_r):
    acc = None
    for h, p_r in enumerate((p0_r, p1_r, p2_r, p3_r)):
        pp = p_r[...]
        hh = pp[0] + pp[1] + b1_r[h, :][None, :]
        t = jnp.dot(hh, w2_r[h], preferred_element_type=jnp.float32)
        acc = t if acc is None else acc + t
    zp01_r[...] = acc[:, :2 * N_CLASSES]
    zp23_r[...] = acc[:, 2 * N_CLASSES:]


def _tc2b_body(zp01_r, zp23_r, al_r, ar_r, el_r, er_r, m_r):
    acc = jnp.concatenate([zp01_r[...], zp23_r[...]], axis=1)
    el_r[...], er_r[...], m_r[...] = _attn_outputs(acc, al_r, ar_r, N_CLASSES)


def _tc3_body(q01_r, q23_r, b2_r, out_r):
    qa = q01_r[...]
    qb = q23_r[...]
    s = qa[0] + qa[1] + qb[0] + qb[1]
    bm = jnp.mean(b2_r[...], axis=0)
    out_r[...] = (s[:, :N_CLASSES] + s[:, N_CLASSES:]) * (1.0 / H) \
        + bm[None, :]


def _sc_logits_body(src_r, dst_r, el0_r, el1_r, el2_r, el3_r,
                    er0_r, er1_r, er2_r, er3_r, m_r,
                    ex0_r, ex1_r, ex2_r, ex3_r,
                    d00_r, d01_r, d10_r, d11_r, d20_r, d21_r, d30_r, d31_r,
                    el_t0, el_t1, el_t2, el_t3, er_t0, er_t1, er_t2, er_t3,
                    m_t, src_blk, dst_blk, exg0, exg1, exg2, exg3, db,
                    den_sh0, den_sh1, den_sh2, den_sh3):
    el_ts = (el_t0, el_t1, el_t2, el_t3)
    er_ts = (er_t0, er_t1, er_t2, er_t3)
    exgs = (exg0, exg1, exg2, exg3)
    ex_rs = (ex0_r, ex1_r, ex2_r, ex3_r)
    den_shs = (den_sh0, den_sh1, den_sh2, den_sh3)
    den_rs = ((d00_r, d01_r), (d10_r, d11_r), (d20_r, d21_r), (d30_r, d31_r))

    c = lax.axis_index("c")
    s = lax.axis_index("s")

    def zgrp(l, _):
        db[pl.ds(l * 16, 16)] = jnp.zeros((16,), jnp.float32)
        return 0
    lax.fori_loop(0, 40, zgrp, 0)
    for h in range(H):
        @pl.when(s < 15)
        def _(h=h):
            pltpu.sync_copy(db, den_shs[h].at[pl.ds(s * 640, 640)])

        @pl.when(s == 15)
        def _(h=h):
            pltpu.sync_copy(db.at[pl.ds(0, 400)],
                            den_shs[h].at[pl.ds(9600, 400)])

    el_in = (el0_r, el1_r, el2_r, el3_r)
    er_in = (er0_r, er1_r, er2_r, er3_r)
    for h in range(H):
        pltpu.sync_copy(el_in[h], el_ts[h])
        pltpu.sync_copy(er_in[h], er_ts[h])
    pltpu.sync_copy(m_r, m_t)
    plsc.subcore_barrier()

    wid = s * 2 + c
    rbase = wid * TROWS
    mvs = [m_t[h, :] for h in range(H)]

    def grp(g, _):
        pltpu.sync_copy(src_r.at[pl.ds(rbase + g * GRP, GRP)], src_blk)
        pltpu.sync_copy(dst_r.at[pl.ds(rbase + g * GRP, GRP)], dst_blk)
        for j in range(GRP):
            for off in OFFS:
                sv = src_blk[j, pl.ds(off, 16)]
                dv = dst_blk[j, pl.ds(off, 16)]
                for h in range(H):
                    t = (plsc.load_gather(el_ts[h], [sv])
                         + plsc.load_gather(er_ts[h], [dv]))
                    ev = jnp.where(t > 0, t, NEG_SLOPE * t)
                    exgs[h][j, pl.ds(off, 16)] = jnp.exp(ev - mvs[h])
        for h in range(H):
            pltpu.sync_copy(exgs[h], ex_rs[h].at[pl.ds(rbase + g * GRP, GRP)])
            for j in range(GRP):
                pltpu.sync_copy(exgs[h].at[j],
                                den_shs[h].at[dst_blk.at[j]], add=True)
        return 0
    lax.fori_loop(0, NGRP, grp, 0)

    plsc.subcore_barrier()

    for cc in range(2):
        @pl.when(c == cc)
        def _(cc=cc):
            for h in range(H):
                @pl.when(s < 15)
                def _(h=h, cc=cc):
                    pltpu.sync_copy(den_shs[h].at[pl.ds(s * 640, 640)], db)
                    pltpu.sync_copy(db, den_rs[h][cc].at[pl.ds(s * 640, 640)])

                @pl.when(s == 15)
                def _(h=h, cc=cc):
                    pltpu.sync_copy(den_shs[h].at[pl.ds(9600, 400)],
                                    db.at[pl.ds(0, 400)])
                    pltpu.sync_copy(db.at[pl.ds(0, 400)],
                                    den_rs[h][cc].at[pl.ds(9600, 400)])


_sc_logits = pl.kernel(
    _sc_logits_body,
    out_type=(
        [jax.ShapeDtypeStruct((ROWS, C), jnp.float32) for _ in range(H)]
        + [jax.ShapeDtypeStruct((N,), jnp.float32) for _ in range(2 * H)]
    ),
    mesh=plsc.VectorSubcoreMesh(core_axis_name="c", subcore_axis_name="s"),
    scratch_types=(
        [pltpu.VMEM((N,), jnp.float32) for _ in range(2 * H)]
        + [
            pltpu.VMEM((H, 16), jnp.float32),
            pltpu.VMEM((GRP, C), jnp.int32),
            pltpu.VMEM((GRP, C), jnp.int32),
        ]
        + [pltpu.VMEM((GRP, C), jnp.float32) for _ in range(H)]
        + [pltpu.VMEM((640,), jnp.float32)]
        + [pltpu.VMEM_SHARED((N,), jnp.float32) for _ in range(H)]
    ),
    compiler_params=pltpu.CompilerParams(needs_layout_passes=False),
)


def _make_sc_msg(NH):
    DH = DT // NH

    def body(*refs):
        (src_r, dst_r, z_r) = refs[0:3]
        ex_rs = refs[3:3 + NH]
        den_rs = refs[3 + NH:3 + 3 * NH]
        out_r = refs[3 + 3 * NH]
        k = 4 + 3 * NH
        (src_blk, dst_blk, rb0, rb1, den_ca, den_cb, inv_c) = refs[k:k + 7]
        k += 7
        exgs = refs[k:k + NH]; k += NH
        acc_sh = refs[k]; k += 1
        (sem0, sem1) = refs[k:k + 2]

        c = lax.axis_index("c")
        s = lax.axis_index("s")

        def zrow(i, _):
            for r in range(DT // 16):
                rb0[i, pl.ds(r * 16, 16)] = jnp.zeros((16,), jnp.float32)
            return 0
        lax.fori_loop(0, 80, zrow, 0)
        nch0 = jnp.where(s == 15, 5, 8)

        def zch(kk, _):
            pltpu.sync_copy(rb0.at[pl.ds(0, 80)],
                            acc_sh.at[pl.ds(s * 640 + kk * 80, 80)])
            return 0
        lax.fori_loop(0, nch0, zch, 0)

        plsc.subcore_barrier()

        wid = s * 2 + c
        rbase = wid * TROWS

        def grp(g, _):
            pltpu.sync_copy(src_r.at[pl.ds(rbase + g * GRP, GRP)], src_blk)
            pltpu.sync_copy(dst_r.at[pl.ds(rbase + g * GRP, GRP)], dst_blk)
            for h in range(NH):
                pltpu.sync_copy(ex_rs[h].at[pl.ds(rbase + g * GRP, GRP)],
                                exgs[h])
            descs = {0: pltpu.async_copy(z_r.at[src_blk.at[0]], rb0, sem0)}
            for j in range(GRP):
                rb = rb0 if j % 2 == 0 else rb1
                descs[j].wait()
                if j + 1 < GRP:
                    nrb, nsem = (rb1, sem1) if j % 2 == 0 else (rb0, sem0)
                    descs[j + 1] = pltpu.async_copy(
                        z_r.at[src_blk.at[j + 1]], nrb, nsem)

                def rbody(i, _2, rb=rb, j=j):
                    ji = jnp.full((16,), j, jnp.int32)
                    bidx = jnp.full((16,), i, jnp.int32)
                    for h in range(NH):
                        av = plsc.load_gather(exgs[h], [ji, bidx])
                        for r in range(DH // 16):
                            q = h * DH + r * 16
                            rb[i, pl.ds(q, 16)] = rb[i, pl.ds(q, 16)] * av
                    return 0
                lax.fori_loop(0, C, rbody, 0)
                pltpu.sync_copy(rb, acc_sh.at[dst_blk.at[j]], add=True)
            return 0
        lax.fori_loop(0, NGRP, grp, 0)

        plsc.subcore_barrier()

        nch = jnp.where(s == 15, 5, 8)

        def p6_ch(kk, _):
            r0 = s * 640 + kk * 80
            pltpu.sync_copy(acc_sh.at[pl.ds(r0, 80)], rb0.at[pl.ds(0, 80)])
            for h in range(NH):
                pltpu.sync_copy(den_rs[2 * h].at[pl.ds(r0, 80)], den_ca)
                pltpu.sync_copy(den_rs[2 * h + 1].at[pl.ds(r0, 80)], den_cb)
                for l in range(5):
                    dv = den_ca[pl.ds(l * 16, 16)] + den_cb[pl.ds(l * 16, 16)]
                    inv_c[h, pl.ds(l * 16, 16)] = 1.0 / (dv + 1e-16)

            def rbody(i, _2):
                bidx = jnp.full((16,), i, jnp.int32)
                for h in range(NH):
                    iv = plsc.load_gather(inv_c.at[h], [bidx])
                    for r in range(DH // 16):
                        q = h * DH + r * 16
                        rb0[i, pl.ds(q, 16)] = rb0[i, pl.ds(q, 16)] * iv
                return 0
            lax.fori_loop(0, 80, rbody, 0)
            pltpu.sync_copy(rb0.at[pl.ds(0, 80)], out_r.at[c, pl.ds(r0, 80)])
            return 0
        lax.fori_loop(0, nch, p6_ch, 0)

    return pl.kernel(
        body,
        out_type=jax.ShapeDtypeStruct((2, N, DT), jnp.float32),
        mesh=plsc.VectorSubcoreMesh(core_axis_name="c", subcore_axis_name="s"),
        scratch_types=(
            [
                pltpu.VMEM((GRP, C), jnp.int32),
                pltpu.VMEM((GRP, C), jnp.int32),
                pltpu.VMEM((C, DT), jnp.float32),
                pltpu.VMEM((C, DT), jnp.float32),
                pltpu.VMEM((80,), jnp.float32),
                pltpu.VMEM((80,), jnp.float32),
                pltpu.VMEM((NH, C), jnp.float32),
            ]
            + [pltpu.VMEM((GRP, C), jnp.float32) for _ in range(NH)]
            + [
                pltpu.VMEM_SHARED((N, DT), jnp.float32),
                pltpu.SemaphoreType.DMA,
                pltpu.SemaphoreType.DMA,
            ]
        ),
        compiler_params=pltpu.CompilerParams(needs_layout_passes=False),
    )


_sc_msg_l1 = _make_sc_msg(1)
_sc_msg_l2 = _make_sc_msg(2)

_tc1 = pl.pallas_call(
    _tc1_body,
    out_shape=(
        [jax.ShapeDtypeStruct((N, D_HID), jnp.float32) for _ in range(H)]
        + [jax.ShapeDtypeStruct((H, N), jnp.float32) for _ in range(2)]
        + [jax.ShapeDtypeStruct((H, 16), jnp.float32)]
    ),
)

_BN2 = 2000

_tc2a = pl.pallas_call(
    _tc2a_body,
    grid=(N // _BN2,),
    in_specs=(
        [pl.BlockSpec((2, _BN2, DT), lambda i: (0, i, 0)) for _ in range(H)]
        + [pl.BlockSpec((H, D_HID), lambda i: (0, 0)),
           pl.BlockSpec((H, D_HID, H * N_CLASSES), lambda i: (0, 0, 0))]
    ),
    out_specs=[pl.BlockSpec((_BN2, 2 * N_CLASSES), lambda i: (i, 0))
               for _ in range(2)],
    out_shape=[jax.ShapeDtypeStruct((N, 2 * N_CLASSES), jnp.float32)
               for _ in range(2)],
)

_tc2b = pl.pallas_call(
    _tc2b_body,
    out_shape=(
        [jax.ShapeDtypeStruct((H, N), jnp.float32) for _ in range(2)]
        + [jax.ShapeDtypeStruct((H, 16), jnp.float32)]
    ),
)

_tc3 = pl.pallas_call(
    _tc3_body,
    out_shape=jax.ShapeDtypeStruct((N, N_CLASSES), jnp.float32),
)


def kernel(x, edge_index, W1, attn_l1, attn_r1, b1, W2, attn_l2, attn_r2, b2):
    src = edge_index[0].reshape(ROWS, C)
    dst = edge_index[1].reshape(ROWS, C)

    *z1, el1, er1, m1 = _tc1(x, W1, attn_l1, attn_r1)
    ex1 = _sc_logits(src, dst, el1[0], el1[1], el1[2], el1[3],
                     er1[0], er1[1], er1[2], er1[3], m1)
    exs1, dens1 = ex1[:H], ex1[H:]
    parts1 = [
        _sc_msg_l1(src, dst, z1[h], exs1[h],
                   dens1[2 * h], dens1[2 * h + 1])
        for h in range(H)
    ]
    zp01, zp23 = _tc2a(*parts1, b1, W2.reshape(H, D_HID, H * N_CLASSES))
    el2, er2, m2 = _tc2b(zp01, zp23, attn_l2, attn_r2)
    ex2 = _sc_logits(src, dst, el2[0], el2[1], el2[2], el2[3],
                     er2[0], er2[1], er2[2], er2[3], m2)
    exs2, dens2 = ex2[:H], ex2[H:]
    q01 = _sc_msg_l2(src, dst, zp01, exs2[0], exs2[1],
                     dens2[0], dens2[1], dens2[2], dens2[3])
    q23 = _sc_msg_l2(src, dst, zp23, exs2[2], exs2[3],
                     dens2[4], dens2[5], dens2[6], dens2[7])
    return _tc3(q01, q23, b2)


# speedup vs baseline: 65.7501x; 1.2610x over previous
import jax
import jax.numpy as jnp
from jax import lax
from jax.experimental import pallas as pl
from jax.experimental.pallas import tpu as pltpu
from jax.experimental.pallas import tpu_sc as plsc

N = 10000
E = 640000
H = 4
D_HID = 128
N_CLASSES = 64
DT = 128

C = 100
ROWS = E // C
TROWS = ROWS // 32
GRP = 8
NGRP = TROWS // GRP
OFFS = (0, 16, 32, 48, 64, 80, 84)
NEG_SLOPE = 0.2


def _attn_outputs(z_cols, al_r, ar_r, dh):
    els, ers = [], []
    for h in range(H):
        zh = z_cols[:, h * dh:(h + 1) * dh]
        els.append(jnp.sum(zh * al_r[h, :][None, :], axis=1))
        ers.append(jnp.sum(zh * ar_r[h, :][None, :], axis=1))
    ms = [jnp.maximum(t, NEG_SLOPE * t)
          for t in [jnp.max(els[h]) + jnp.max(ers[h]) for h in range(H)]]
    mbc = jnp.broadcast_to(jnp.stack(ms, axis=0)[:, None], (H, 16))
    return jnp.stack(els, axis=0), jnp.stack(ers, axis=0), mbc


def _tc1_body(x_r, w_r, al_r, ar_r, z0_r, z1_r, z2_r, z3_r,
              el_r, er_r, m_r):
    z = jnp.dot(x_r[...], w_r[...], preferred_element_type=jnp.float32)
    zrefs = (z0_r, z1_r, z2_r, z3_r)
    for h in range(H):
        zrefs[h][...] = z[:, h * D_HID:(h + 1) * D_HID]
    el_r[...], er_r[...], m_r[...] = _attn_outputs(z, al_r, ar_r, D_HID)


def _tc2a_body(p0_r, p1_r, p2_r, p3_r, b1_r, w2_r, zp01_r, zp23_r):
    acc = None
    for h, p_r in enumerate((p0_r, p1_r, p2_r, p3_r)):
        pp = p_r[...]
        hh = pp[0] + pp[1] + b1_r[h, :][None, :]
        t = jnp.dot(hh, w2_r[h], preferred_element_type=jnp.float32)
        acc = t if acc is None else acc + t
    zp01_r[...] = acc[:, :2 * N_CLASSES]
    zp23_r[...] = acc[:, 2 * N_CLASSES:]


def _tc2b_body(zp01_r, zp23_r, al_r, ar_r, el_r, er_r, m_r):
    acc = jnp.concatenate([zp01_r[...], zp23_r[...]], axis=1)
    el_r[...], er_r[...], m_r[...] = _attn_outputs(acc, al_r, ar_r, N_CLASSES)


def _tc3_body(q01_r, q23_r, b2_r, out_r):
    qa = q01_r[...]
    qb = q23_r[...]
    s = qa[0] + qa[1] + qb[0] + qb[1]
    bm = jnp.mean(b2_r[...], axis=0)
    out_r[...] = (s[:, :N_CLASSES] + s[:, N_CLASSES:]) * (1.0 / H) \
        + bm[None, :]


def _sc_logits_body(src_r, dst_r, el0_r, el1_r, el2_r, el3_r,
                    er0_r, er1_r, er2_r, er3_r, m_r,
                    ex0_r, ex1_r, ex2_r, ex3_r,
                    d00_r, d01_r, d10_r, d11_r, d20_r, d21_r, d30_r, d31_r,
                    el_t0, el_t1, el_t2, el_t3, er_t0, er_t1, er_t2, er_t3,
                    m_t, src_blk, dst_blk, exg0, exg1, exg2, exg3, db,
                    den_sh0, den_sh1, den_sh2, den_sh3, semA, semB):
    el_ts = (el_t0, el_t1, el_t2, el_t3)
    er_ts = (er_t0, er_t1, er_t2, er_t3)
    exgs = (exg0, exg1, exg2, exg3)
    ex_rs = (ex0_r, ex1_r, ex2_r, ex3_r)
    den_shs = (den_sh0, den_sh1, den_sh2, den_sh3)
    den_rs = ((d00_r, d01_r), (d10_r, d11_r), (d20_r, d21_r), (d30_r, d31_r))

    c = lax.axis_index("c")
    s = lax.axis_index("s")

    def zgrp(l, _):
        db[pl.ds(l * 16, 16)] = jnp.zeros((16,), jnp.float32)
        return 0
    lax.fori_loop(0, 40, zgrp, 0)
    for h in range(H):
        @pl.when(s < 15)
        def _(h=h):
            pltpu.sync_copy(db, den_shs[h].at[pl.ds(s * 640, 640)])

        @pl.when(s == 15)
        def _(h=h):
            pltpu.sync_copy(db.at[pl.ds(0, 400)],
                            den_shs[h].at[pl.ds(9600, 400)])

    el_in = (el0_r, el1_r, el2_r, el3_r)
    er_in = (er0_r, er1_r, er2_r, er3_r)
    for h in range(H):
        pltpu.sync_copy(el_in[h], el_ts[h])
        pltpu.sync_copy(er_in[h], er_ts[h])
    pltpu.sync_copy(m_r, m_t)
    plsc.subcore_barrier()

    wid = s * 2 + c
    rbase = wid * TROWS
    mvs = [m_t[h, :] for h in range(H)]

    def grp(g, _):
        pltpu.sync_copy(src_r.at[pl.ds(rbase + g * GRP, GRP)], src_blk)
        pltpu.sync_copy(dst_r.at[pl.ds(rbase + g * GRP, GRP)], dst_blk)
        for j in range(GRP):
            for off in OFFS:
                sv = src_blk[j, pl.ds(off, 16)]
                dv = dst_blk[j, pl.ds(off, 16)]
                for h in range(H):
                    t = (plsc.load_gather(el_ts[h], [sv])
                         + plsc.load_gather(er_ts[h], [dv]))
                    ev = jnp.where(t > 0, t, NEG_SLOPE * t)
                    exgs[h][j, pl.ds(off, 16)] = jnp.exp(ev - mvs[h])
        descs = []
        for h in range(H):
            descs.append(pltpu.async_copy(
                exgs[h], ex_rs[h].at[pl.ds(rbase + g * GRP, GRP)], semA))
            for j in range(GRP):
                descs.append(pltpu.async_copy(
                    exgs[h].at[j], den_shs[h].at[dst_blk.at[j]], semB,
                    add=True))
        for dd in descs:
            dd.wait()
        return 0
    lax.fori_loop(0, NGRP, grp, 0)

    plsc.subcore_barrier()

    for cc in range(2):
        @pl.when(c == cc)
        def _(cc=cc):
            for h in range(H):
                @pl.when(s < 15)
                def _(h=h, cc=cc):
                    pltpu.sync_copy(den_shs[h].at[pl.ds(s * 640, 640)], db)
                    pltpu.sync_copy(db, den_rs[h][cc].at[pl.ds(s * 640, 640)])

                @pl.when(s == 15)
                def _(h=h, cc=cc):
                    pltpu.sync_copy(den_shs[h].at[pl.ds(9600, 400)],
                                    db.at[pl.ds(0, 400)])
                    pltpu.sync_copy(db.at[pl.ds(0, 400)],
                                    den_rs[h][cc].at[pl.ds(9600, 400)])


_sc_logits = pl.kernel(
    _sc_logits_body,
    out_type=(
        [jax.ShapeDtypeStruct((ROWS, C), jnp.float32) for _ in range(H)]
        + [jax.ShapeDtypeStruct((N,), jnp.float32) for _ in range(2 * H)]
    ),
    mesh=plsc.VectorSubcoreMesh(core_axis_name="c", subcore_axis_name="s"),
    scratch_types=(
        [pltpu.VMEM((N,), jnp.float32) for _ in range(2 * H)]
        + [
            pltpu.VMEM((H, 16), jnp.float32),
            pltpu.VMEM((GRP, C), jnp.int32),
            pltpu.VMEM((GRP, C), jnp.int32),
        ]
        + [pltpu.VMEM((GRP, C), jnp.float32) for _ in range(H)]
        + [pltpu.VMEM((640,), jnp.float32)]
        + [pltpu.VMEM_SHARED((N,), jnp.float32) for _ in range(H)]
        + [pltpu.SemaphoreType.DMA, pltpu.SemaphoreType.DMA]
    ),
    compiler_params=pltpu.CompilerParams(needs_layout_passes=False),
)


def _make_sc_msg(NH):
    DH = DT // NH

    def body(*refs):
        (src_r, dst_r, z_r) = refs[0:3]
        ex_rs = refs[3:3 + NH]
        den_rs = refs[3 + NH:3 + 3 * NH]
        out_r = refs[3 + 3 * NH]
        k = 4 + 3 * NH
        (src_blk, dst_blk, rb0, rb1, den_ca, den_cb, inv_c) = refs[k:k + 7]
        k += 7
        exgs = refs[k:k + NH]; k += NH
        acc_sh = refs[k]; k += 1
        (sem0, sem1, sem2, sem3) = refs[k:k + 4]

        c = lax.axis_index("c")
        s = lax.axis_index("s")

        @plsc.parallel_loop(0, 80, 1, unroll=4)
        def zrow(i):
            for r in range(DT // 16):
                rb0[i, pl.ds(r * 16, 16)] = jnp.zeros((16,), jnp.float32)
        nch0 = jnp.where(s == 15, 5, 8)

        def zch(kk, _):
            pltpu.sync_copy(rb0.at[pl.ds(0, 80)],
                            acc_sh.at[pl.ds(s * 640 + kk * 80, 80)])
            return 0
        lax.fori_loop(0, nch0, zch, 0)

        plsc.subcore_barrier()

        wid = s * 2 + c
        rbase = wid * TROWS

        def grp(g, _):
            pltpu.sync_copy(src_r.at[pl.ds(rbase + g * GRP, GRP)], src_blk)
            pltpu.sync_copy(dst_r.at[pl.ds(rbase + g * GRP, GRP)], dst_blk)
            for h in range(NH):
                pltpu.sync_copy(ex_rs[h].at[pl.ds(rbase + g * GRP, GRP)],
                                exgs[h])
            gd = {0: pltpu.async_copy(z_r.at[src_blk.at[0]], rb0, sem0)}
            for j in range(GRP):
                rb = rb0 if j % 2 == 0 else rb1
                gd[j].wait()
                if j + 1 < GRP:
                    nrb, nsem = (rb1, sem1) if j % 2 == 0 else (rb0, sem0)
                    gd[j + 1] = pltpu.async_copy(
                        z_r.at[src_blk.at[j + 1]], nrb, nsem)

                @plsc.parallel_loop(0, C, 1, unroll=4)
                def rbody(i, rb=rb, j=j):
                    ji = jnp.full((16,), j, jnp.int32)
                    bidx = jnp.full((16,), i, jnp.int32)
                    for h in range(NH):
                        av = plsc.load_gather(exgs[h], [ji, bidx])
                        for r in range(DH // 16):
                            q = h * DH + r * 16
                            rb[i, pl.ds(q, 16)] = rb[i, pl.ds(q, 16)] * av
                pltpu.sync_copy(rb, acc_sh.at[dst_blk.at[j]], add=True)
            return 0
        lax.fori_loop(0, NGRP, grp, 0)

        plsc.subcore_barrier()

        nch = jnp.where(s == 15, 5, 8)

        def p6_ch(kk, _):
            r0 = s * 640 + kk * 80
            pltpu.sync_copy(acc_sh.at[pl.ds(r0, 80)], rb0.at[pl.ds(0, 80)])
            for h in range(NH):
                pltpu.sync_copy(den_rs[2 * h].at[pl.ds(r0, 80)], den_ca)
                pltpu.sync_copy(den_rs[2 * h + 1].at[pl.ds(r0, 80)], den_cb)
                for l in range(5):
                    dv = den_ca[pl.ds(l * 16, 16)] + den_cb[pl.ds(l * 16, 16)]
                    inv_c[h, pl.ds(l * 16, 16)] = 1.0 / (dv + 1e-16)

            @plsc.parallel_loop(0, 80, 1, unroll=4)
            def rbody(i):
                bidx = jnp.full((16,), i, jnp.int32)
                for h in range(NH):
                    iv = plsc.load_gather(inv_c.at[h], [bidx])
                    for r in range(DH // 16):
                        q = h * DH + r * 16
                        rb0[i, pl.ds(q, 16)] = rb0[i, pl.ds(q, 16)] * iv
            pltpu.sync_copy(rb0.at[pl.ds(0, 80)], out_r.at[c, pl.ds(r0, 80)])
            return 0
        lax.fori_loop(0, nch, p6_ch, 0)

    return pl.kernel(
        body,
        out_type=jax.ShapeDtypeStruct((2, N, DT), jnp.float32),
        mesh=plsc.VectorSubcoreMesh(core_axis_name="c", subcore_axis_name="s"),
        scratch_types=(
            [
                pltpu.VMEM((GRP, C), jnp.int32),
                pltpu.VMEM((GRP, C), jnp.int32),
                pltpu.VMEM((C, DT), jnp.float32),
                pltpu.VMEM((C, DT), jnp.float32),
                pltpu.VMEM((80,), jnp.float32),
                pltpu.VMEM((80,), jnp.float32),
                pltpu.VMEM((NH, C), jnp.float32),
            ]
            + [pltpu.VMEM((GRP, C), jnp.float32) for _ in range(NH)]
            + [
                pltpu.VMEM_SHARED((N, DT), jnp.float32),
                pltpu.SemaphoreType.DMA,
                pltpu.SemaphoreType.DMA,
                pltpu.SemaphoreType.DMA,
                pltpu.SemaphoreType.DMA,
            ]
        ),
        compiler_params=pltpu.CompilerParams(needs_layout_passes=False),
    )


_sc_msg_l1 = _make_sc_msg(1)
_sc_msg_l2 = _make_sc_msg(2)

_tc1 = pl.pallas_call(
    _tc1_body,
    out_shape=(
        [jax.ShapeDtypeStruct((N, D_HID), jnp.float32) for _ in range(H)]
        + [jax.ShapeDtypeStruct((H, N), jnp.float32) for _ in range(2)]
        + [jax.ShapeDtypeStruct((H, 16), jnp.float32)]
    ),
)

_BN2 = 2000

_tc2a = pl.pallas_call(
    _tc2a_body,
    grid=(N // _BN2,),
    in_specs=(
        [pl.BlockSpec((2, _BN2, DT), lambda i: (0, i, 0)) for _ in range(H)]
        + [pl.BlockSpec((H, D_HID), lambda i: (0, 0)),
           pl.BlockSpec((H, D_HID, H * N_CLASSES), lambda i: (0, 0, 0))]
    ),
    out_specs=[pl.BlockSpec((_BN2, 2 * N_CLASSES), lambda i: (i, 0))
               for _ in range(2)],
    out_shape=[jax.ShapeDtypeStruct((N, 2 * N_CLASSES), jnp.float32)
               for _ in range(2)],
)

_tc2b = pl.pallas_call(
    _tc2b_body,
    out_shape=(
        [jax.ShapeDtypeStruct((H, N), jnp.float32) for _ in range(2)]
        + [jax.ShapeDtypeStruct((H, 16), jnp.float32)]
    ),
)

_tc3 = pl.pallas_call(
    _tc3_body,
    out_shape=jax.ShapeDtypeStruct((N, N_CLASSES), jnp.float32),
)


def kernel(x, edge_index, W1, attn_l1, attn_r1, b1, W2, attn_l2, attn_r2, b2):
    src = edge_index[0].reshape(ROWS, C)
    dst = edge_index[1].reshape(ROWS, C)

    *z1, el1, er1, m1 = _tc1(x, W1, attn_l1, attn_r1)
    ex1 = _sc_logits(src, dst, el1[0], el1[1], el1[2], el1[3],
                     er1[0], er1[1], er1[2], er1[3], m1)
    exs1, dens1 = ex1[:H], ex1[H:]
    parts1 = [
        _sc_msg_l1(src, dst, z1[h], exs1[h],
                   dens1[2 * h], dens1[2 * h + 1])
        for h in range(H)
    ]
    zp01, zp23 = _tc2a(*parts1, b1, W2.reshape(H, D_HID, H * N_CLASSES))
    el2, er2, m2 = _tc2b(zp01, zp23, attn_l2, attn_r2)
    ex2 = _sc_logits(src, dst, el2[0], el2[1], el2[2], el2[3],
                     er2[0], er2[1], er2[2], er2[3], m2)
    exs2, dens2 = ex2[:H], ex2[H:]
    q01 = _sc_msg_l2(src, dst, zp01, exs2[0], exs2[1],
                     dens2[0], dens2[1], dens2[2], dens2[3])
    q23 = _sc_msg_l2(src, dst, zp23, exs2[2], exs2[3],
                     dens2[4], dens2[5], dens2[6], dens2[7])
    return _tc3(q01, q23, b2)


# speedup vs baseline: 67.5198x; 1.0269x over previous
import jax
import jax.numpy as jnp
from jax import lax
from jax.experimental import pallas as pl
from jax.experimental.pallas import tpu as pltpu
from jax.experimental.pallas import tpu_sc as plsc

N = 10000
E = 640000
H = 4
D_HID = 128
N_CLASSES = 64
DT = 128

C = 100
ROWS = E // C
TROWS = ROWS // 32
GRP = 8
NGRP = TROWS // GRP
OFFS = (0, 16, 32, 48, 64, 80, 84)
NEG_SLOPE = 0.2


def _attn_outputs(z_cols, al_r, ar_r, dh):
    els, ers = [], []
    for h in range(H):
        zh = z_cols[:, h * dh:(h + 1) * dh]
        els.append(jnp.sum(zh * al_r[h, :][None, :], axis=1))
        ers.append(jnp.sum(zh * ar_r[h, :][None, :], axis=1))
    ms = [jnp.maximum(t, NEG_SLOPE * t)
          for t in [jnp.max(els[h]) + jnp.max(ers[h]) for h in range(H)]]
    mbc = jnp.broadcast_to(jnp.stack(ms, axis=0)[:, None], (H, 16))
    return jnp.stack(els, axis=0), jnp.stack(ers, axis=0), mbc


def _tc1_body(x_r, w_r, al_r, ar_r, z0_r, z1_r, z2_r, z3_r,
              el_r, er_r, m_r):
    z = jnp.dot(x_r[...], w_r[...], preferred_element_type=jnp.float32)
    zrefs = (z0_r, z1_r, z2_r, z3_r)
    for h in range(H):
        zrefs[h][...] = z[:, h * D_HID:(h + 1) * D_HID]
    el_r[...], er_r[...], m_r[...] = _attn_outputs(z, al_r, ar_r, D_HID)


def _tc2a_body(p0_r, p1_r, p2_r, p3_r, b1_r, w2_r, zp01_r, zp23_r):
    acc = None
    for h, p_r in enumerate((p0_r, p1_r, p2_r, p3_r)):
        pp = p_r[...]
        hh = pp[0] + pp[1] + b1_r[h, :][None, :]
        t = jnp.dot(hh, w2_r[h], preferred_element_type=jnp.float32)
        acc = t if acc is None else acc + t
    zp01_r[...] = acc[:, :2 * N_CLASSES]
    zp23_r[...] = acc[:, 2 * N_CLASSES:]


def _tc2b_body(zp01_r, zp23_r, al_r, ar_r, el_r, er_r, m_r):
    acc = jnp.concatenate([zp01_r[...], zp23_r[...]], axis=1)
    el_r[...], er_r[...], m_r[...] = _attn_outputs(acc, al_r, ar_r, N_CLASSES)


def _tc3_body(q01_r, q23_r, b2_r, out_r):
    qa = q01_r[...]
    qb = q23_r[...]
    s = qa[0] + qa[1] + qb[0] + qb[1]
    bm = jnp.mean(b2_r[...], axis=0)
    out_r[...] = (s[:, :N_CLASSES] + s[:, N_CLASSES:]) * (1.0 / H) \
        + bm[None, :]


def _sc_logits_body(src_r, dst_r, el0_r, el1_r, el2_r, el3_r,
                    er0_r, er1_r, er2_r, er3_r, m_r,
                    ex0_r, ex1_r, ex2_r, ex3_r,
                    d00_r, d01_r, d10_r, d11_r, d20_r, d21_r, d30_r, d31_r,
                    el_t0, el_t1, el_t2, el_t3, er_t0, er_t1, er_t2, er_t3,
                    m_t, src_blk, dst_blk, exg0, exg1, exg2, exg3, db,
                    den_sh0, den_sh1, den_sh2, den_sh3, semA, semB):
    el_ts = (el_t0, el_t1, el_t2, el_t3)
    er_ts = (er_t0, er_t1, er_t2, er_t3)
    exgs = (exg0, exg1, exg2, exg3)
    ex_rs = (ex0_r, ex1_r, ex2_r, ex3_r)
    den_shs = (den_sh0, den_sh1, den_sh2, den_sh3)
    den_rs = ((d00_r, d01_r), (d10_r, d11_r), (d20_r, d21_r), (d30_r, d31_r))

    c = lax.axis_index("c")
    s = lax.axis_index("s")

    def zgrp(l, _):
        db[pl.ds(l * 16, 16)] = jnp.zeros((16,), jnp.float32)
        return 0
    lax.fori_loop(0, 40, zgrp, 0)
    for h in range(H):
        @pl.when(s < 15)
        def _(h=h):
            pltpu.sync_copy(db, den_shs[h].at[pl.ds(s * 640, 640)])

        @pl.when(s == 15)
        def _(h=h):
            pltpu.sync_copy(db.at[pl.ds(0, 400)],
                            den_shs[h].at[pl.ds(9600, 400)])

    el_in = (el0_r, el1_r, el2_r, el3_r)
    er_in = (er0_r, er1_r, er2_r, er3_r)
    for h in range(H):
        pltpu.sync_copy(el_in[h], el_ts[h])
        pltpu.sync_copy(er_in[h], er_ts[h])
    pltpu.sync_copy(m_r, m_t)
    plsc.subcore_barrier()

    wid = s * 2 + c
    rbase = wid * TROWS
    mvs = [m_t[h, :] for h in range(H)]

    def grp(g, _):
        pltpu.sync_copy(src_r.at[pl.ds(rbase + g * GRP, GRP)], src_blk)
        pltpu.sync_copy(dst_r.at[pl.ds(rbase + g * GRP, GRP)], dst_blk)
        for j in range(GRP):
            for off in OFFS:
                sv = src_blk[j, pl.ds(off, 16)]
                dv = dst_blk[j, pl.ds(off, 16)]
                for h in range(H):
                    t = (plsc.load_gather(el_ts[h], [sv])
                         + plsc.load_gather(er_ts[h], [dv]))
                    ev = jnp.where(t > 0, t, NEG_SLOPE * t)
                    exgs[h][j, pl.ds(off, 16)] = jnp.exp(ev - mvs[h])
        descs = []
        for h in range(H):
            descs.append(pltpu.async_copy(
                exgs[h], ex_rs[h].at[pl.ds(rbase + g * GRP, GRP)], semA))
            for j in range(GRP):
                descs.append(pltpu.async_copy(
                    exgs[h].at[j], den_shs[h].at[dst_blk.at[j]], semB,
                    add=True))
        for dd in descs:
            dd.wait()
        return 0
    lax.fori_loop(0, NGRP, grp, 0)

    plsc.subcore_barrier()

    for cc in range(2):
        @pl.when(c == cc)
        def _(cc=cc):
            for h in range(H):
                @pl.when(s < 15)
                def _(h=h, cc=cc):
                    pltpu.sync_copy(den_shs[h].at[pl.ds(s * 640, 640)], db)
                    pltpu.sync_copy(db, den_rs[h][cc].at[pl.ds(s * 640, 640)])

                @pl.when(s == 15)
                def _(h=h, cc=cc):
                    pltpu.sync_copy(den_shs[h].at[pl.ds(9600, 400)],
                                    db.at[pl.ds(0, 400)])
                    pltpu.sync_copy(db.at[pl.ds(0, 400)],
                                    den_rs[h][cc].at[pl.ds(9600, 400)])


_sc_logits = pl.kernel(
    _sc_logits_body,
    out_type=(
        [jax.ShapeDtypeStruct((ROWS, C), jnp.float32) for _ in range(H)]
        + [jax.ShapeDtypeStruct((N,), jnp.float32) for _ in range(2 * H)]
    ),
    mesh=plsc.VectorSubcoreMesh(core_axis_name="c", subcore_axis_name="s"),
    scratch_types=(
        [pltpu.VMEM((N,), jnp.float32) for _ in range(2 * H)]
        + [
            pltpu.VMEM((H, 16), jnp.float32),
            pltpu.VMEM((GRP, C), jnp.int32),
            pltpu.VMEM((GRP, C), jnp.int32),
        ]
        + [pltpu.VMEM((GRP, C), jnp.float32) for _ in range(H)]
        + [pltpu.VMEM((640,), jnp.float32)]
        + [pltpu.VMEM_SHARED((N,), jnp.float32) for _ in range(H)]
        + [pltpu.SemaphoreType.DMA, pltpu.SemaphoreType.DMA]
    ),
    compiler_params=pltpu.CompilerParams(needs_layout_passes=False),
)


def _make_sc_msg(NH):
    DH = DT // NH

    def body(*refs):
        (src_r, dst_r, z_r) = refs[0:3]
        ex_rs = refs[3:3 + NH]
        den_rs = refs[3 + NH:3 + 3 * NH]
        out_r = refs[3 + 3 * NH]
        k = 4 + 3 * NH
        (src_blk, dst_blk, rb0, rb1, rb2, den_ca, den_cb, inv_c) = \
            refs[k:k + 8]
        k += 8
        exgs = refs[k:k + NH]; k += NH
        acc_sh = refs[k]; k += 1
        (sem0, sem1, sem2) = refs[k:k + 3]
        rbs = (rb0, rb1, rb2)
        sems = (sem0, sem1, sem2)

        c = lax.axis_index("c")
        s = lax.axis_index("s")

        @plsc.parallel_loop(0, 80, 1, unroll=4)
        def zrow(i):
            for r in range(DT // 16):
                rb0[i, pl.ds(r * 16, 16)] = jnp.zeros((16,), jnp.float32)
        nch0 = jnp.where(s == 15, 5, 8)

        def zch(kk, _):
            pltpu.sync_copy(rb0.at[pl.ds(0, 80)],
                            acc_sh.at[pl.ds(s * 640 + kk * 80, 80)])
            return 0
        lax.fori_loop(0, nch0, zch, 0)

        plsc.subcore_barrier()

        wid = s * 2 + c
        rbase = wid * TROWS

        def grp(g, _):
            pltpu.sync_copy(src_r.at[pl.ds(rbase + g * GRP, GRP)], src_blk)
            pltpu.sync_copy(dst_r.at[pl.ds(rbase + g * GRP, GRP)], dst_blk)
            for h in range(NH):
                pltpu.sync_copy(ex_rs[h].at[pl.ds(rbase + g * GRP, GRP)],
                                exgs[h])
            gd = {
                0: pltpu.async_copy(z_r.at[src_blk.at[0]], rbs[0], sems[0]),
                1: pltpu.async_copy(z_r.at[src_blk.at[1]], rbs[1], sems[1]),
            }
            for j in range(GRP):
                rb = rbs[j % 3]
                gd[j].wait()
                if j + 2 < GRP:
                    b = (j + 2) % 3
                    gd[j + 2] = pltpu.async_copy(
                        z_r.at[src_blk.at[j + 2]], rbs[b], sems[b])

                @plsc.parallel_loop(0, C, 1, unroll=4)
                def rbody(i, rb=rb, j=j):
                    ji = jnp.full((16,), j, jnp.int32)
                    bidx = jnp.full((16,), i, jnp.int32)
                    for h in range(NH):
                        av = plsc.load_gather(exgs[h], [ji, bidx])
                        for r in range(DH // 16):
                            q = h * DH + r * 16
                            rb[i, pl.ds(q, 16)] = rb[i, pl.ds(q, 16)] * av
                pltpu.sync_copy(rb, acc_sh.at[dst_blk.at[j]], add=True)
            return 0
        lax.fori_loop(0, NGRP, grp, 0)

        plsc.subcore_barrier()

        nch = jnp.where(s == 15, 5, 8)

        def p6_ch(kk, _):
            r0 = s * 640 + kk * 80
            pltpu.sync_copy(acc_sh.at[pl.ds(r0, 80)], rb0.at[pl.ds(0, 80)])
            for h in range(NH):
                pltpu.sync_copy(den_rs[2 * h].at[pl.ds(r0, 80)], den_ca)
                pltpu.sync_copy(den_rs[2 * h + 1].at[pl.ds(r0, 80)], den_cb)
                for l in range(5):
                    dv = den_ca[pl.ds(l * 16, 16)] + den_cb[pl.ds(l * 16, 16)]
                    inv_c[h, pl.ds(l * 16, 16)] = 1.0 / (dv + 1e-16)

            @plsc.parallel_loop(0, 80, 1, unroll=4)
            def rbody(i):
                bidx = jnp.full((16,), i, jnp.int32)
                for h in range(NH):
                    iv = plsc.load_gather(inv_c.at[h], [bidx])
                    for r in range(DH // 16):
                        q = h * DH + r * 16
                        rb0[i, pl.ds(q, 16)] = rb0[i, pl.ds(q, 16)] * iv
            pltpu.sync_copy(rb0.at[pl.ds(0, 80)], out_r.at[c, pl.ds(r0, 80)])
            return 0
        lax.fori_loop(0, nch, p6_ch, 0)

    return pl.kernel(
        body,
        out_type=jax.ShapeDtypeStruct((2, N, DT), jnp.float32),
        mesh=plsc.VectorSubcoreMesh(core_axis_name="c", subcore_axis_name="s"),
        scratch_types=(
            [
                pltpu.VMEM((GRP, C), jnp.int32),
                pltpu.VMEM((GRP, C), jnp.int32),
                pltpu.VMEM((C, DT), jnp.float32),
                pltpu.VMEM((C, DT), jnp.float32),
                pltpu.VMEM((C, DT), jnp.float32),
                pltpu.VMEM((80,), jnp.float32),
                pltpu.VMEM((80,), jnp.float32),
                pltpu.VMEM((NH, C), jnp.float32),
            ]
            + [pltpu.VMEM((GRP, C), jnp.float32) for _ in range(NH)]
            + [
                pltpu.VMEM_SHARED((N, DT), jnp.float32),
                pltpu.SemaphoreType.DMA,
                pltpu.SemaphoreType.DMA,
                pltpu.SemaphoreType.DMA,
            ]
        ),
        compiler_params=pltpu.CompilerParams(needs_layout_passes=False),
    )


_sc_msg_l1 = _make_sc_msg(1)
_sc_msg_l2 = _make_sc_msg(2)

_tc1 = pl.pallas_call(
    _tc1_body,
    out_shape=(
        [jax.ShapeDtypeStruct((N, D_HID), jnp.float32) for _ in range(H)]
        + [jax.ShapeDtypeStruct((H, N), jnp.float32) for _ in range(2)]
        + [jax.ShapeDtypeStruct((H, 16), jnp.float32)]
    ),
)

_BN2 = 2000

_tc2a = pl.pallas_call(
    _tc2a_body,
    grid=(N // _BN2,),
    in_specs=(
        [pl.BlockSpec((2, _BN2, DT), lambda i: (0, i, 0)) for _ in range(H)]
        + [pl.BlockSpec((H, D_HID), lambda i: (0, 0)),
           pl.BlockSpec((H, D_HID, H * N_CLASSES), lambda i: (0, 0, 0))]
    ),
    out_specs=[pl.BlockSpec((_BN2, 2 * N_CLASSES), lambda i: (i, 0))
               for _ in range(2)],
    out_shape=[jax.ShapeDtypeStruct((N, 2 * N_CLASSES), jnp.float32)
               for _ in range(2)],
)

_tc2b = pl.pallas_call(
    _tc2b_body,
    out_shape=(
        [jax.ShapeDtypeStruct((H, N), jnp.float32) for _ in range(2)]
        + [jax.ShapeDtypeStruct((H, 16), jnp.float32)]
    ),
)

_tc3 = pl.pallas_call(
    _tc3_body,
    out_shape=jax.ShapeDtypeStruct((N, N_CLASSES), jnp.float32),
)


def kernel(x, edge_index, W1, attn_l1, attn_r1, b1, W2, attn_l2, attn_r2, b2):
    src = edge_index[0].reshape(ROWS, C)
    dst = edge_index[1].reshape(ROWS, C)

    *z1, el1, er1, m1 = _tc1(x, W1, attn_l1, attn_r1)
    ex1 = _sc_logits(src, dst, el1[0], el1[1], el1[2], el1[3],
                     er1[0], er1[1], er1[2], er1[3], m1)
    exs1, dens1 = ex1[:H], ex1[H:]
    parts1 = [
        _sc_msg_l1(src, dst, z1[h], exs1[h],
                   dens1[2 * h], dens1[2 * h + 1])
        for h in range(H)
    ]
    zp01, zp23 = _tc2a(*parts1, b1, W2.reshape(H, D_HID, H * N_CLASSES))
    el2, er2, m2 = _tc2b(zp01, zp23, attn_l2, attn_r2)
    ex2 = _sc_logits(src, dst, el2[0], el2[1], el2[2], el2[3],
                     er2[0], er2[1], er2[2], er2[3], m2)
    exs2, dens2 = ex2[:H], ex2[H:]
    q01 = _sc_msg_l2(src, dst, zp01, exs2[0], exs2[1],
                     dens2[0], dens2[1], dens2[2], dens2[3])
    q23 = _sc_msg_l2(src, dst, zp23, exs2[2], exs2[3],
                     dens2[4], dens2[5], dens2[6], dens2[7])
    return _tc3(q01, q23, b2)


# speedup vs baseline: 69.5514x; 1.0301x over previous
import jax
import jax.numpy as jnp
from jax import lax
from jax.experimental import pallas as pl
from jax.experimental.pallas import tpu as pltpu
from jax.experimental.pallas import tpu_sc as plsc

N = 10000
E = 640000
H = 4
D_HID = 128
N_CLASSES = 64
DT = 128

C = 100
ROWS = E // C
TROWS = ROWS // 32
GRP = 8
NGRP = TROWS // GRP
OFFS = (0, 16, 32, 48, 64, 80, 84)
NEG_SLOPE = 0.2


def _attn_outputs(z_cols, al_r, ar_r, dh):
    els, ers = [], []
    for h in range(H):
        zh = z_cols[:, h * dh:(h + 1) * dh]
        els.append(jnp.sum(zh * al_r[h, :][None, :], axis=1))
        ers.append(jnp.sum(zh * ar_r[h, :][None, :], axis=1))
    ms = [jnp.maximum(t, NEG_SLOPE * t)
          for t in [jnp.max(els[h]) + jnp.max(ers[h]) for h in range(H)]]
    mbc = jnp.broadcast_to(jnp.stack(ms, axis=0)[:, None], (H, 16))
    return jnp.stack(els, axis=0), jnp.stack(ers, axis=0), mbc


def _tc1_body(x_r, w_r, al_r, ar_r, z0_r, z1_r, z2_r, z3_r,
              el_r, er_r, m_r):
    z = jnp.dot(x_r[...], w_r[...], preferred_element_type=jnp.float32)
    zrefs = (z0_r, z1_r, z2_r, z3_r)
    for h in range(H):
        zrefs[h][...] = z[:, h * D_HID:(h + 1) * D_HID]
    el_r[...], er_r[...], m_r[...] = _attn_outputs(z, al_r, ar_r, D_HID)


def _tc2a_body(p0_r, p1_r, p2_r, p3_r, b1_r, w2_r, zp01_r, zp23_r):
    acc = None
    for h, p_r in enumerate((p0_r, p1_r, p2_r, p3_r)):
        pp = p_r[...]
        hh = pp[0] + pp[1] + b1_r[h, :][None, :]
        t = jnp.dot(hh, w2_r[h], preferred_element_type=jnp.float32)
        acc = t if acc is None else acc + t
    zp01_r[...] = acc[:, :2 * N_CLASSES]
    zp23_r[...] = acc[:, 2 * N_CLASSES:]


def _tc2b_body(zp01_r, zp23_r, al_r, ar_r, el_r, er_r, m_r):
    acc = jnp.concatenate([zp01_r[...], zp23_r[...]], axis=1)
    el_r[...], er_r[...], m_r[...] = _attn_outputs(acc, al_r, ar_r, N_CLASSES)


def _tc3_body(q01_r, q23_r, b2_r, out_r):
    qa = q01_r[...]
    qb = q23_r[...]
    s = qa[0] + qa[1] + qb[0] + qb[1]
    bm = jnp.mean(b2_r[...], axis=0)
    out_r[...] = (s[:, :N_CLASSES] + s[:, N_CLASSES:]) * (1.0 / H) \
        + bm[None, :]


def _sc_logits_body(src_r, dst_r, el0_r, el1_r, el2_r, el3_r,
                    er0_r, er1_r, er2_r, er3_r, m_r,
                    ex0_r, ex1_r, ex2_r, ex3_r,
                    d00_r, d01_r, d10_r, d11_r, d20_r, d21_r, d30_r, d31_r,
                    el_t0, el_t1, el_t2, el_t3, er_t0, er_t1, er_t2, er_t3,
                    m_t, src_blk0, dst_blk0, exg00, exg01, exg02, exg03,
                    src_blk1, dst_blk1, exg10, exg11, exg12, exg13, db,
                    den_sh0, den_sh1, den_sh2, den_sh3,
                    semA0, semB0, semA1, semB1):
    el_ts = (el_t0, el_t1, el_t2, el_t3)
    er_ts = (er_t0, er_t1, er_t2, er_t3)
    sets = (
        (src_blk0, dst_blk0, (exg00, exg01, exg02, exg03), semA0, semB0),
        (src_blk1, dst_blk1, (exg10, exg11, exg12, exg13), semA1, semB1),
    )
    ex_rs = (ex0_r, ex1_r, ex2_r, ex3_r)
    den_shs = (den_sh0, den_sh1, den_sh2, den_sh3)
    den_rs = ((d00_r, d01_r), (d10_r, d11_r), (d20_r, d21_r), (d30_r, d31_r))

    c = lax.axis_index("c")
    s = lax.axis_index("s")

    def zgrp(l, _):
        db[pl.ds(l * 16, 16)] = jnp.zeros((16,), jnp.float32)
        return 0
    lax.fori_loop(0, 40, zgrp, 0)
    for h in range(H):
        @pl.when(s < 15)
        def _(h=h):
            pltpu.sync_copy(db, den_shs[h].at[pl.ds(s * 640, 640)])

        @pl.when(s == 15)
        def _(h=h):
            pltpu.sync_copy(db.at[pl.ds(0, 400)],
                            den_shs[h].at[pl.ds(9600, 400)])

    el_in = (el0_r, el1_r, el2_r, el3_r)
    er_in = (er0_r, er1_r, er2_r, er3_r)
    for h in range(H):
        pltpu.sync_copy(el_in[h], el_ts[h])
        pltpu.sync_copy(er_in[h], er_ts[h])
    pltpu.sync_copy(m_r, m_t)
    plsc.subcore_barrier()

    wid = s * 2 + c
    rbase = wid * TROWS
    mvs = [m_t[h, :] for h in range(H)]

    def fire_group(g, pset):
        src_blk, dst_blk, exgs, semA, semB = sets[pset]
        pltpu.sync_copy(src_r.at[pl.ds(rbase + g * GRP, GRP)], src_blk)
        pltpu.sync_copy(dst_r.at[pl.ds(rbase + g * GRP, GRP)], dst_blk)

        def cj(j, _):
            for off in OFFS:
                sv = src_blk[j, pl.ds(off, 16)]
                dv = dst_blk[j, pl.ds(off, 16)]
                for h in range(H):
                    t = (plsc.load_gather(el_ts[h], [sv])
                         + plsc.load_gather(er_ts[h], [dv]))
                    ev = jnp.where(t > 0, t, NEG_SLOPE * t)
                    exgs[h][j, pl.ds(off, 16)] = jnp.exp(ev - mvs[h])
            return 0
        lax.fori_loop(0, GRP, cj, 0)
        descs = []
        for h in range(H):
            descs.append(pltpu.async_copy(
                exgs[h], ex_rs[h].at[pl.ds(rbase + g * GRP, GRP)], semA))
            for j in range(GRP):
                descs.append(pltpu.async_copy(
                    exgs[h].at[j], den_shs[h].at[dst_blk.at[j]], semB,
                    add=True))
        return descs

    def pair(p, _):
        d0 = fire_group(2 * p, 0)
        d1 = fire_group(2 * p + 1, 1)
        for dd in d0 + d1:
            dd.wait()
        return 0
    lax.fori_loop(0, NGRP // 2, pair, 0)
    for dd in fire_group(NGRP - 1, 0):
        dd.wait()

    plsc.subcore_barrier()

    for cc in range(2):
        @pl.when(c == cc)
        def _(cc=cc):
            for h in range(H):
                @pl.when(s < 15)
                def _(h=h, cc=cc):
                    pltpu.sync_copy(den_shs[h].at[pl.ds(s * 640, 640)], db)
                    pltpu.sync_copy(db, den_rs[h][cc].at[pl.ds(s * 640, 640)])

                @pl.when(s == 15)
                def _(h=h, cc=cc):
                    pltpu.sync_copy(den_shs[h].at[pl.ds(9600, 400)],
                                    db.at[pl.ds(0, 400)])
                    pltpu.sync_copy(db.at[pl.ds(0, 400)],
                                    den_rs[h][cc].at[pl.ds(9600, 400)])


_sc_logits = pl.kernel(
    _sc_logits_body,
    out_type=(
        [jax.ShapeDtypeStruct((ROWS, C), jnp.float32) for _ in range(H)]
        + [jax.ShapeDtypeStruct((N,), jnp.float32) for _ in range(2 * H)]
    ),
    mesh=plsc.VectorSubcoreMesh(core_axis_name="c", subcore_axis_name="s"),
    scratch_types=(
        [pltpu.VMEM((N,), jnp.float32) for _ in range(2 * H)]
        + [pltpu.VMEM((H, 16), jnp.float32)]
        + [
            pltpu.VMEM((GRP, C), jnp.int32),
            pltpu.VMEM((GRP, C), jnp.int32),
            pltpu.VMEM((GRP, C), jnp.float32),
            pltpu.VMEM((GRP, C), jnp.float32),
            pltpu.VMEM((GRP, C), jnp.float32),
            pltpu.VMEM((GRP, C), jnp.float32),
        ] * 2
        + [pltpu.VMEM((640,), jnp.float32)]
        + [pltpu.VMEM_SHARED((N,), jnp.float32) for _ in range(H)]
        + [pltpu.SemaphoreType.DMA] * 4
    ),
    compiler_params=pltpu.CompilerParams(needs_layout_passes=False),
)


def _make_sc_msg(NH):
    DH = DT // NH

    def body(*refs):
        (src_r, dst_r, z_r) = refs[0:3]
        ex_rs = refs[3:3 + NH]
        den_rs = refs[3 + NH:3 + 3 * NH]
        out_r = refs[3 + 3 * NH]
        k = 4 + 3 * NH
        (src_blk, dst_blk, rb0, rb1, rb2, den_ca, den_cb, inv_c) = \
            refs[k:k + 8]
        k += 8
        exgs = refs[k:k + NH]; k += NH
        acc_sh = refs[k]; k += 1
        (sem0, sem1, sem2) = refs[k:k + 3]
        rbs = (rb0, rb1, rb2)
        sems = (sem0, sem1, sem2)

        c = lax.axis_index("c")
        s = lax.axis_index("s")

        @plsc.parallel_loop(0, 80, 1, unroll=4)
        def zrow(i):
            for r in range(DT // 16):
                rb0[i, pl.ds(r * 16, 16)] = jnp.zeros((16,), jnp.float32)
        nch0 = jnp.where(s == 15, 5, 8)

        def zch(kk, _):
            pltpu.sync_copy(rb0.at[pl.ds(0, 80)],
                            acc_sh.at[pl.ds(s * 640 + kk * 80, 80)])
            return 0
        lax.fori_loop(0, nch0, zch, 0)

        plsc.subcore_barrier()

        wid = s * 2 + c
        rbase = wid * TROWS

        def grp(g, _):
            pltpu.sync_copy(src_r.at[pl.ds(rbase + g * GRP, GRP)], src_blk)
            pltpu.sync_copy(dst_r.at[pl.ds(rbase + g * GRP, GRP)], dst_blk)
            for h in range(NH):
                pltpu.sync_copy(ex_rs[h].at[pl.ds(rbase + g * GRP, GRP)],
                                exgs[h])
            gd = {
                0: pltpu.async_copy(z_r.at[src_blk.at[0]], rbs[0], sems[0]),
                1: pltpu.async_copy(z_r.at[src_blk.at[1]], rbs[1], sems[1]),
            }
            for j in range(GRP):
                rb = rbs[j % 3]
                gd[j].wait()
                if j + 2 < GRP:
                    b = (j + 2) % 3
                    gd[j + 2] = pltpu.async_copy(
                        z_r.at[src_blk.at[j + 2]], rbs[b], sems[b])

                @plsc.parallel_loop(0, C, 1, unroll=4)
                def rbody(i, rb=rb, j=j):
                    ji = jnp.full((16,), j, jnp.int32)
                    bidx = jnp.full((16,), i, jnp.int32)
                    for h in range(NH):
                        av = plsc.load_gather(exgs[h], [ji, bidx])
                        for r in range(DH // 16):
                            q = h * DH + r * 16
                            rb[i, pl.ds(q, 16)] = rb[i, pl.ds(q, 16)] * av
                pltpu.sync_copy(rb, acc_sh.at[dst_blk.at[j]], add=True)
            return 0
        lax.fori_loop(0, NGRP, grp, 0)

        plsc.subcore_barrier()

        nch = jnp.where(s == 15, 5, 8)

        def p6_ch(kk, _):
            r0 = s * 640 + kk * 80
            pltpu.sync_copy(acc_sh.at[pl.ds(r0, 80)], rb0.at[pl.ds(0, 80)])
            for h in range(NH):
                pltpu.sync_copy(den_rs[2 * h].at[pl.ds(r0, 80)], den_ca)
                pltpu.sync_copy(den_rs[2 * h + 1].at[pl.ds(r0, 80)], den_cb)
                for l in range(5):
                    dv = den_ca[pl.ds(l * 16, 16)] + den_cb[pl.ds(l * 16, 16)]
                    inv_c[h, pl.ds(l * 16, 16)] = 1.0 / (dv + 1e-16)

            @plsc.parallel_loop(0, 80, 1, unroll=4)
            def rbody(i):
                bidx = jnp.full((16,), i, jnp.int32)
                for h in range(NH):
                    iv = plsc.load_gather(inv_c.at[h], [bidx])
                    for r in range(DH // 16):
                        q = h * DH + r * 16
                        rb0[i, pl.ds(q, 16)] = rb0[i, pl.ds(q, 16)] * iv
            pltpu.sync_copy(rb0.at[pl.ds(0, 80)], out_r.at[c, pl.ds(r0, 80)])
            return 0
        lax.fori_loop(0, nch, p6_ch, 0)

    return pl.kernel(
        body,
        out_type=jax.ShapeDtypeStruct((2, N, DT), jnp.float32),
        mesh=plsc.VectorSubcoreMesh(core_axis_name="c", subcore_axis_name="s"),
        scratch_types=(
            [
                pltpu.VMEM((GRP, C), jnp.int32),
                pltpu.VMEM((GRP, C), jnp.int32),
                pltpu.VMEM((C, DT), jnp.float32),
                pltpu.VMEM((C, DT), jnp.float32),
                pltpu.VMEM((C, DT), jnp.float32),
                pltpu.VMEM((80,), jnp.float32),
                pltpu.VMEM((80,), jnp.float32),
                pltpu.VMEM((NH, C), jnp.float32),
            ]
            + [pltpu.VMEM((GRP, C), jnp.float32) for _ in range(NH)]
            + [
                pltpu.VMEM_SHARED((N, DT), jnp.float32),
                pltpu.SemaphoreType.DMA,
                pltpu.SemaphoreType.DMA,
                pltpu.SemaphoreType.DMA,
            ]
        ),
        compiler_params=pltpu.CompilerParams(needs_layout_passes=False),
    )


_sc_msg_l1 = _make_sc_msg(1)
_sc_msg_l2 = _make_sc_msg(2)

_tc1 = pl.pallas_call(
    _tc1_body,
    out_shape=(
        [jax.ShapeDtypeStruct((N, D_HID), jnp.float32) for _ in range(H)]
        + [jax.ShapeDtypeStruct((H, N), jnp.float32) for _ in range(2)]
        + [jax.ShapeDtypeStruct((H, 16), jnp.float32)]
    ),
)

_BN2 = 2000

_tc2a = pl.pallas_call(
    _tc2a_body,
    grid=(N // _BN2,),
    in_specs=(
        [pl.BlockSpec((2, _BN2, DT), lambda i: (0, i, 0)) for _ in range(H)]
        + [pl.BlockSpec((H, D_HID), lambda i: (0, 0)),
           pl.BlockSpec((H, D_HID, H * N_CLASSES), lambda i: (0, 0, 0))]
    ),
    out_specs=[pl.BlockSpec((_BN2, 2 * N_CLASSES), lambda i: (i, 0))
               for _ in range(2)],
    out_shape=[jax.ShapeDtypeStruct((N, 2 * N_CLASSES), jnp.float32)
               for _ in range(2)],
)

_tc2b = pl.pallas_call(
    _tc2b_body,
    out_shape=(
        [jax.ShapeDtypeStruct((H, N), jnp.float32) for _ in range(2)]
        + [jax.ShapeDtypeStruct((H, 16), jnp.float32)]
    ),
)

_tc3 = pl.pallas_call(
    _tc3_body,
    out_shape=jax.ShapeDtypeStruct((N, N_CLASSES), jnp.float32),
)


def kernel(x, edge_index, W1, attn_l1, attn_r1, b1, W2, attn_l2, attn_r2, b2):
    src = edge_index[0].reshape(ROWS, C)
    dst = edge_index[1].reshape(ROWS, C)

    *z1, el1, er1, m1 = _tc1(x, W1, attn_l1, attn_r1)
    ex1 = _sc_logits(src, dst, el1[0], el1[1], el1[2], el1[3],
                     er1[0], er1[1], er1[2], er1[3], m1)
    exs1, dens1 = ex1[:H], ex1[H:]
    parts1 = [
        _sc_msg_l1(src, dst, z1[h], exs1[h],
                   dens1[2 * h], dens1[2 * h + 1])
        for h in range(H)
    ]
    zp01, zp23 = _tc2a(*parts1, b1, W2.reshape(H, D_HID, H * N_CLASSES))
    el2, er2, m2 = _tc2b(zp01, zp23, attn_l2, attn_r2)
    ex2 = _sc_logits(src, dst, el2[0], el2[1], el2[2], el2[3],
                     er2[0], er2[1], er2[2], er2[3], m2)
    exs2, dens2 = ex2[:H], ex2[H:]
    q01 = _sc_msg_l2(src, dst, zp01, exs2[0], exs2[1],
                     dens2[0], dens2[1], dens2[2], dens2[3])
    q23 = _sc_msg_l2(src, dst, zp23, exs2[2], exs2[3],
                     dens2[4], dens2[5], dens2[6], dens2[7])
    return _tc3(q01, q23, b2)
